# baseline (speedup 1.0000x reference)
import jax
import jax.numpy as jnp
from jax.experimental import pallas as pl
from jax.experimental.pallas import tpu as pltpu

_B, _T, _D, _H = 8, 2048, 256, 256
_CHUNK = 32
_NCH = _T // _CHUNK


def _gru2_kernel(x_ref, wih0_ref, whh0_ref, bih0_ref, bhh0_ref,
                 wih1_ref, whh1_ref, bih1_ref, bhh1_ref,
                 y_ref, h0_ref, h1_ref, gi0_ref, gi1_ref, y0_ref):
    c = pl.program_id(0)

    @pl.when(c == 0)
    def _init0():
        h0_ref[...] = jnp.zeros_like(h0_ref)

    @pl.when(c <= 1)
    def _init1():
        h1_ref[...] = jnp.zeros_like(h1_ref)

    gi0_ref[...] = (
        jnp.dot(x_ref[...], wih0_ref[...], preferred_element_type=jnp.float32)
        + bih0_ref[...]
    )

    def gates(g_i, g_h, h):
        r = jax.nn.sigmoid(g_i[:, :_H] + g_h[:, :_H])
        z = jax.nn.sigmoid(g_i[:, _H:2 * _H] + g_h[:, _H:2 * _H])
        n = jnp.tanh(g_i[:, 2 * _H:] + r * g_h[:, 2 * _H:])
        return (1.0 - z) * n + z * h

    def layer_step(h, w_ref, b_ref, gi):
        hb16 = h.astype(jnp.bfloat16)
        gh_r = jnp.dot(hb16, w_ref[:, :_H],
                       preferred_element_type=jnp.float32) + b_ref[:, :_H]
        gh_z = jnp.dot(hb16, w_ref[:, _H:2 * _H],
                       preferred_element_type=jnp.float32) + b_ref[:, _H:2 * _H]
        gh_n = jnp.dot(hb16, w_ref[:, 2 * _H:],
                       preferred_element_type=jnp.float32) + b_ref[:, 2 * _H:]
        r = jax.nn.sigmoid(gi[:, :_H] + gh_r)
        z = jax.nn.sigmoid(gi[:, _H:2 * _H] + gh_z)
        n = jnp.tanh(gi[:, 2 * _H:] + r * gh_n)
        return (1.0 - z) * n + z * h

    def body(i, carry):
        h0, h1 = carry
        h0_next = layer_step(h0, whh0_ref, bhh0_ref, gi0_ref[pl.ds(i * _B, _B)])
        h1_next = layer_step(h1, whh1_ref, bhh1_ref, gi1_ref[pl.ds(i * _B, _B)])
        y0_ref[pl.ds(i * _B, _B)] = h0_next
        y_ref[pl.ds(i * _B, _B)] = h1_next
        return h0_next, h1_next

    h0, h1 = jax.lax.fori_loop(0, _CHUNK, body, (h0_ref[...], h1_ref[...]),
                               unroll=64)
    h0_ref[...] = h0
    h1_ref[...] = h1

    gi1_ref[...] = (
        jnp.dot(y0_ref[...].astype(jnp.bfloat16), wih1_ref[...],
                preferred_element_type=jnp.float32)
        + bih1_ref[...]
    )


@jax.jit
def kernel(x, w_ih_l0, w_hh_l0, b_ih_l0, b_hh_l0,
           w_ih_l1, w_hh_l1, b_ih_l1, b_hh_l1):
    xt = jnp.swapaxes(x, 0, 1).reshape(_T * _B, _D).astype(jnp.bfloat16)

    full = lambda shape: pl.BlockSpec(shape, lambda c: (0,) * len(shape))
    y = pl.pallas_call(
        _gru2_kernel,
        grid=(_NCH + 1,),
        in_specs=[
            pl.BlockSpec((_CHUNK * _B, _D),
                         lambda c: (jnp.minimum(c, _NCH - 1), 0)),
            full((_D, 3 * _H)),
            full((_H, 3 * _H)),
            full((1, 3 * _H)),
            full((1, 3 * _H)),
            full((_H, 3 * _H)),
            full((_H, 3 * _H)),
            full((1, 3 * _H)),
            full((1, 3 * _H)),
        ],
        out_specs=pl.BlockSpec((_CHUNK * _B, _H),
                               lambda c: (jnp.maximum(c - 1, 0), 0)),
        out_shape=jax.ShapeDtypeStruct((_T * _B, _H), jnp.float32),
        scratch_shapes=[
            pltpu.VMEM((_B, _H), jnp.float32),
            pltpu.VMEM((_B, _H), jnp.float32),
            pltpu.VMEM((_CHUNK * _B, 3 * _H), jnp.float32),
            pltpu.VMEM((_CHUNK * _B, 3 * _H), jnp.float32),
            pltpu.VMEM((_CHUNK * _B, _H), jnp.float32),
        ],
        compiler_params=pltpu.CompilerParams(
            dimension_semantics=("arbitrary",),
        ),
    )(
        xt,
        w_ih_l0.T.astype(jnp.bfloat16), w_hh_l0.T.astype(jnp.bfloat16),
        b_ih_l0[None], b_hh_l0[None],
        w_ih_l1.T.astype(jnp.bfloat16), w_hh_l1.T.astype(jnp.bfloat16),
        b_ih_l1[None], b_hh_l1[None],
    )
    return jnp.swapaxes(y.reshape(_T, _B, _H), 0, 1)


# speedup vs baseline: 1.0173x; 1.0173x over previous
import jax
import jax.numpy as jnp
from jax.experimental import pallas as pl
from jax.experimental.pallas import tpu as pltpu

_B, _T, _D, _H = 8, 2048, 256, 256
_CHUNK = 64
_NCH = _T // _CHUNK


def _gru2_kernel(x_ref, wih0_ref, whh0_ref, bih0_ref, bhh0_ref,
                 wih1_ref, whh1_ref, bih1_ref, bhh1_ref,
                 y_ref, h0_ref, h1_ref, gi0_ref, gi1_ref, y0_ref):
    c = pl.program_id(0)

    @pl.when(c == 0)
    def _init0():
        h0_ref[...] = jnp.zeros_like(h0_ref)

    @pl.when(c <= 1)
    def _init1():
        h1_ref[...] = jnp.zeros_like(h1_ref)

    gi0_ref[...] = (
        jnp.dot(x_ref[...], wih0_ref[...], preferred_element_type=jnp.float32)
        + bih0_ref[...]
    )

    def gates(g_i, g_h, h):
        r = jax.nn.sigmoid(g_i[:, :_H] + g_h[:, :_H])
        z = jax.nn.sigmoid(g_i[:, _H:2 * _H] + g_h[:, _H:2 * _H])
        n = jnp.tanh(g_i[:, 2 * _H:] + r * g_h[:, 2 * _H:])
        return (1.0 - z) * n + z * h

    def layer_step(h, w_ref, b_ref, gi):
        hb16 = h.astype(jnp.bfloat16)
        gh_r = jnp.dot(hb16, w_ref[:, :_H],
                       preferred_element_type=jnp.float32) + b_ref[:, :_H]
        gh_z = jnp.dot(hb16, w_ref[:, _H:2 * _H],
                       preferred_element_type=jnp.float32) + b_ref[:, _H:2 * _H]
        gh_n = jnp.dot(hb16, w_ref[:, 2 * _H:],
                       preferred_element_type=jnp.float32) + b_ref[:, 2 * _H:]
        r = jax.nn.sigmoid(gi[:, :_H] + gh_r)
        z = jax.nn.sigmoid(gi[:, _H:2 * _H] + gh_z)
        n = jnp.tanh(gi[:, 2 * _H:] + r * gh_n)
        return (1.0 - z) * n + z * h

    def body(i, carry):
        h0, h1 = carry
        h0_next = layer_step(h0, whh0_ref, bhh0_ref, gi0_ref[pl.ds(i * _B, _B)])
        h1_next = layer_step(h1, whh1_ref, bhh1_ref, gi1_ref[pl.ds(i * _B, _B)])
        y0_ref[pl.ds(i * _B, _B)] = h0_next
        y_ref[pl.ds(i * _B, _B)] = h1_next
        return h0_next, h1_next

    h0, h1 = jax.lax.fori_loop(0, _CHUNK, body, (h0_ref[...], h1_ref[...]),
                               unroll=64)
    h0_ref[...] = h0
    h1_ref[...] = h1

    gi1_ref[...] = (
        jnp.dot(y0_ref[...].astype(jnp.bfloat16), wih1_ref[...],
                preferred_element_type=jnp.float32)
        + bih1_ref[...]
    )


@jax.jit
def kernel(x, w_ih_l0, w_hh_l0, b_ih_l0, b_hh_l0,
           w_ih_l1, w_hh_l1, b_ih_l1, b_hh_l1):
    xt = jnp.swapaxes(x, 0, 1).reshape(_T * _B, _D).astype(jnp.bfloat16)

    full = lambda shape: pl.BlockSpec(shape, lambda c: (0,) * len(shape))
    y = pl.pallas_call(
        _gru2_kernel,
        grid=(_NCH + 1,),
        in_specs=[
            pl.BlockSpec((_CHUNK * _B, _D),
                         lambda c: (jnp.minimum(c, _NCH - 1), 0)),
            full((_D, 3 * _H)),
            full((_H, 3 * _H)),
            full((1, 3 * _H)),
            full((1, 3 * _H)),
            full((_H, 3 * _H)),
            full((_H, 3 * _H)),
            full((1, 3 * _H)),
            full((1, 3 * _H)),
        ],
        out_specs=pl.BlockSpec((_CHUNK * _B, _H),
                               lambda c: (jnp.maximum(c - 1, 0), 0)),
        out_shape=jax.ShapeDtypeStruct((_T * _B, _H), jnp.float32),
        scratch_shapes=[
            pltpu.VMEM((_B, _H), jnp.float32),
            pltpu.VMEM((_B, _H), jnp.float32),
            pltpu.VMEM((_CHUNK * _B, 3 * _H), jnp.float32),
            pltpu.VMEM((_CHUNK * _B, 3 * _H), jnp.float32),
            pltpu.VMEM((_CHUNK * _B, _H), jnp.float32),
        ],
        compiler_params=pltpu.CompilerParams(
            dimension_semantics=("arbitrary",),
        ),
    )(
        xt,
        w_ih_l0.T.astype(jnp.bfloat16), w_hh_l0.T.astype(jnp.bfloat16),
        b_ih_l0[None], b_hh_l0[None],
        w_ih_l1.T.astype(jnp.bfloat16), w_hh_l1.T.astype(jnp.bfloat16),
        b_ih_l1[None], b_hh_l1[None],
    )
    return jnp.swapaxes(y.reshape(_T, _B, _H), 0, 1)
